# baseline (speedup 1.0000x reference)
import dataclasses
import functools

import jax
import jax.numpy as jnp
import numpy as np
from jax import lax
from jax.experimental import pallas as pl
from jax.experimental.pallas import tpu as pltpu
from jax.experimental.pallas import tpu_sc as plsc

_NC, _NS = 2, 16
_NW = _NC * _NS
_C = 128
_NB = 5


@functools.partial(jax.jit, static_argnums=(3, 4, 5))
def _sc_gather(table, flat_idx, slen, n, d, l):
    n_per_w = n // _NW
    nch = n_per_w // _C
    nchunks = n // _C
    assert n_per_w % _C == 0 and nch % _NB == 0

    mesh = plsc.VectorSubcoreMesh(core_axis_name="c", subcore_axis_name="s")

    cp = pltpu.CompilerParams()
    if "needs_layout_passes" in pltpu.CompilerParams.__dataclass_fields__:
        cp = dataclasses.replace(cp, needs_layout_passes=False)

    @functools.partial(
        pl.kernel,
        out_type=jax.ShapeDtypeStruct((n, d), table.dtype),
        mesh=mesh,
        compiler_params=cp,
        scratch_types=[
            pltpu.VMEM((n // _NW,), jnp.int32),
            pltpu.VMEM((_NB, _C, d), table.dtype),
            pltpu.VMEM((slen.shape[0],), jnp.int32),
            pltpu.VMEM((_C, d), table.dtype),
            pltpu.SemaphoreType.DMA((_NB,)),
            pltpu.SemaphoreType.DMA((_NB,)),
        ],
    )
    def gather_kernel(table_hbm, idx_hbm, slen_hbm, out_hbm,
                      idx_v, rows_v, slen_v, zero_v, gsem, osem):
        wid = lax.axis_index("s") * _NC + lax.axis_index("c")
        base = wid * n_per_w

        pltpu.sync_copy(idx_hbm.at[pl.ds(base, n_per_w)], idx_v)
        pltpu.sync_copy(slen_hbm, slen_v)
        lane = lax.iota(jnp.int32, 16)
        zvec = jnp.zeros((16,), table.dtype)

        def _sample_scalar(off16):
            iv = jnp.max(jnp.full((16,), off16, jnp.int32) // l)
            win = slen_v[pl.ds((iv // 16) * 16, 16)]
            ln = jnp.max(jnp.where(lane == (iv & 15), win, 0))
            return iv, ln

        @pl.loop(0, _C)
        def _(r):
            for j in range(d // 16):
                zero_v[r, pl.ds(j * 16, 16)] = zvec

        @pl.loop(0, nch, step=_NB)
        def _(k):
            scal = []
            for p in range(_NB):
                off = base + (k + p) * _C

                @pl.when(k + p >= _NB)
                def _():
                    pltpu.make_async_copy(
                        rows_v.at[p],
                        out_hbm.at[pl.ds(off - _NB * _C, _C)],
                        osem.at[p],
                    ).wait()

                i0, len0 = _sample_scalar(off)
                i1, len1 = _sample_scalar(off + _C - 1)
                a1 = jnp.clip(i0 * l + len0 - off, 0, _C)
                b1 = jnp.clip((i0 + 1) * l - off, 0, _C)
                a2 = jnp.where(i1 == i0, 0, jnp.clip(i1 * l + len1 - off, 0, _C))
                b2 = jnp.where(i1 == i0, 0, jnp.clip((i1 + 1) * l - off, 0, _C))
                live = jnp.logical_or(a1 > 0, b1 < _C)
                scal.append((a1, b1, a2, b2, live))

                @pl.when(live)
                def _():
                    pltpu.make_async_copy(
                        table_hbm.at[idx_v.at[pl.ds((k + p) * _C, _C)]],
                        rows_v.at[p], gsem.at[p],
                    ).start()

            for p in range(_NB):
                off = base + (k + p) * _C
                a1, b1, a2, b2, live = scal[p]

                @pl.when(live)
                def _():
                    pltpu.make_async_copy(
                        table_hbm.at[idx_v.at[pl.ds((k + p) * _C, _C)]],
                        rows_v.at[p], gsem.at[p],
                    ).wait()

                    for a, b in ((a1, b1), (a2, b2)):

                        @pl.loop(a, b)
                        def _(r):
                            for j in range(d // 16):
                                rows_v[p, r, pl.ds(j * 16, 16)] = zvec

                    pltpu.make_async_copy(
                        rows_v.at[p], out_hbm.at[pl.ds(off, _C)], osem.at[p]
                    ).start()

                @pl.when(jnp.logical_not(live))
                def _():
                    pltpu.make_async_copy(
                        zero_v, out_hbm.at[pl.ds(off, _C)], osem.at[p]
                    ).start()

        for p in range(_NB):
            off = base + (nch - _NB + p) * _C
            pltpu.make_async_copy(
                rows_v.at[p], out_hbm.at[pl.ds(off, _C)], osem.at[p]
            ).wait()

    return gather_kernel(table, flat_idx, slen)


def kernel(input_batch, seq_lengths, targets_batch, table):
    B, L = input_batch.shape
    V, D = table.shape
    n = B * L
    nchunks = n // _C

    lengths = jnp.maximum(seq_lengths, 1).astype(jnp.int32)
    perm = jnp.argsort(-lengths)
    slen = lengths[perm]

    flat_idx = input_batch[perm].astype(jnp.int32).reshape(n)

    embedded = _sc_gather(table, flat_idx, slen, n, D, L).reshape(B, L, D)
    return embedded, slen.astype(jnp.float32), targets_batch[perm]


# speedup vs baseline: 1.0348x; 1.0348x over previous
import dataclasses
import functools

import jax
import jax.numpy as jnp
import numpy as np
from jax import lax
from jax.experimental import pallas as pl
from jax.experimental.pallas import tpu as pltpu
from jax.experimental.pallas import tpu_sc as plsc

_NC, _NS = 2, 16
_NW = _NC * _NS
_C = 64
_NB = 10


@functools.partial(jax.jit, static_argnums=(3, 4, 5))
def _sc_gather(table, flat_idx, slen, n, d, l):
    n_per_w = n // _NW
    nch = n_per_w // _C
    nchunks = n // _C
    assert n_per_w % _C == 0 and nch % _NB == 0

    mesh = plsc.VectorSubcoreMesh(core_axis_name="c", subcore_axis_name="s")

    cp = pltpu.CompilerParams()
    if "needs_layout_passes" in pltpu.CompilerParams.__dataclass_fields__:
        cp = dataclasses.replace(cp, needs_layout_passes=False)

    @functools.partial(
        pl.kernel,
        out_type=jax.ShapeDtypeStruct((n, d), table.dtype),
        mesh=mesh,
        compiler_params=cp,
        scratch_types=[
            pltpu.VMEM((n // _NW,), jnp.int32),
            pltpu.VMEM((_NB, _C, d), table.dtype),
            pltpu.VMEM((slen.shape[0],), jnp.int32),
            pltpu.VMEM((_C, d), table.dtype),
            pltpu.SemaphoreType.DMA((_NB,)),
            pltpu.SemaphoreType.DMA((_NB,)),
        ],
    )
    def gather_kernel(table_hbm, idx_hbm, slen_hbm, out_hbm,
                      idx_v, rows_v, slen_v, zero_v, gsem, osem):
        wid = lax.axis_index("s") * _NC + lax.axis_index("c")
        base = wid * n_per_w

        pltpu.sync_copy(idx_hbm.at[pl.ds(base, n_per_w)], idx_v)
        pltpu.sync_copy(slen_hbm, slen_v)
        lane = lax.iota(jnp.int32, 16)
        zvec = jnp.zeros((16,), table.dtype)

        def _sample_scalar(off16):
            iv = jnp.max(jnp.full((16,), off16, jnp.int32) // l)
            win = slen_v[pl.ds((iv // 16) * 16, 16)]
            ln = jnp.max(jnp.where(lane == (iv & 15), win, 0))
            return iv, ln

        @pl.loop(0, _C)
        def _(r):
            for j in range(d // 16):
                zero_v[r, pl.ds(j * 16, 16)] = zvec

        @pl.loop(0, nch, step=_NB)
        def _(k):
            scal = []
            for p in range(_NB):
                off = base + (k + p) * _C

                @pl.when(k + p >= _NB)
                def _():
                    pltpu.make_async_copy(
                        rows_v.at[p],
                        out_hbm.at[pl.ds(off - _NB * _C, _C)],
                        osem.at[p],
                    ).wait()

                i0, len0 = _sample_scalar(off)
                i1, len1 = _sample_scalar(off + _C - 1)
                a1 = jnp.clip(i0 * l + len0 - off, 0, _C)
                b1 = jnp.clip((i0 + 1) * l - off, 0, _C)
                a2 = jnp.where(i1 == i0, 0, jnp.clip(i1 * l + len1 - off, 0, _C))
                b2 = jnp.where(i1 == i0, 0, jnp.clip((i1 + 1) * l - off, 0, _C))
                live = jnp.logical_or(a1 > 0, b1 < _C)
                scal.append((a1, b1, a2, b2, live))

                @pl.when(live)
                def _():
                    pltpu.make_async_copy(
                        table_hbm.at[idx_v.at[pl.ds((k + p) * _C, _C)]],
                        rows_v.at[p], gsem.at[p],
                    ).start()

            for p in range(_NB):
                off = base + (k + p) * _C
                a1, b1, a2, b2, live = scal[p]

                @pl.when(live)
                def _():
                    pltpu.make_async_copy(
                        table_hbm.at[idx_v.at[pl.ds((k + p) * _C, _C)]],
                        rows_v.at[p], gsem.at[p],
                    ).wait()

                    for a, b in ((a1, b1), (a2, b2)):

                        @pl.loop(a, b)
                        def _(r):
                            for j in range(d // 16):
                                rows_v[p, r, pl.ds(j * 16, 16)] = zvec

                    pltpu.make_async_copy(
                        rows_v.at[p], out_hbm.at[pl.ds(off, _C)], osem.at[p]
                    ).start()

                @pl.when(jnp.logical_not(live))
                def _():
                    pltpu.make_async_copy(
                        zero_v, out_hbm.at[pl.ds(off, _C)], osem.at[p]
                    ).start()

        for p in range(_NB):
            off = base + (nch - _NB + p) * _C
            pltpu.make_async_copy(
                rows_v.at[p], out_hbm.at[pl.ds(off, _C)], osem.at[p]
            ).wait()

    return gather_kernel(table, flat_idx, slen)


def kernel(input_batch, seq_lengths, targets_batch, table):
    B, L = input_batch.shape
    V, D = table.shape
    n = B * L
    nchunks = n // _C

    lengths = jnp.maximum(seq_lengths, 1).astype(jnp.int32)
    perm = jnp.argsort(-lengths)
    slen = lengths[perm]

    flat_idx = input_batch[perm].astype(jnp.int32).reshape(n)

    embedded = _sc_gather(table, flat_idx, slen, n, D, L).reshape(B, L, D)
    return embedded, slen.astype(jnp.float32), targets_batch[perm]
